# baseline (speedup 1.0000x reference)
import functools
import numpy as np
import jax
import jax.numpy as jnp
from jax.experimental import pallas as pl
from jax.experimental.pallas import tpu as pltpu

_EPS = np.float32(216.0 / 24389.0)
_KAPPA = np.float32(24389.0 / 27.0)
_NBINS = 256
_GRID = 8


def _dx(x, y):
    q = x / y
    return q + (x - q * y) / y


def _srgb_to_linear(c):
    u = _dx(c + 0.055, np.float32(1.055))
    p = jnp.exp(jnp.log(u) * np.float32(2.4))
    return jnp.where(c <= 0.04045, c / 12.92, p)


def _cbrt(t):
    return jnp.exp(jnp.log(t) * np.float32(1.0 / 3.0))


def _lab_f(t):
    return jnp.where(t > _EPS, _cbrt(t),
                     _dx(_KAPPA * t + 16.0, np.float32(116.0)))


def _stats_kernel(x_ref, lq_ref, a_ref, b_ref, luts_ref, lsc, hist_sc):
    H = W = 512
    TH = 64
    SLAB = 64
    mins = []
    maxs = []
    for si in range(H // SLAB):
        sl = slice(si * SLAB, (si + 1) * SLAB)
        r = x_ref[0, 0, sl, :]
        g = x_ref[0, 1, sl, :]
        bch = x_ref[0, 2, sl, :]
        lr = _srgb_to_linear(r)
        lg = _srgb_to_linear(g)
        lb = _srgb_to_linear(bch)
        xx = (lr * 0.4124564 + lg * 0.3575761 + lb * 0.1804375) / 0.95047
        yy = lr * 0.2126729 + lg * 0.7151522 + lb * 0.0721750
        zz = (lr * 0.0193339 + lg * 0.1191920 + lb * 0.9503041) / 1.08883
        fx = _lab_f(xx)
        fy = _lab_f(yy)
        fz = _lab_f(zz)
        L = 116.0 * fy - 16.0
        a_ref[0, sl, :] = 500.0 * (fx - fy)
        b_ref[0, sl, :] = 200.0 * (fy - fz)
        lsc[sl, :] = L
        mins.append(jnp.min(L, axis=(0, 1), keepdims=True))
        maxs.append(jnp.max(L, axis=(0, 1), keepdims=True))
    lmin = functools.reduce(jnp.minimum, mins)
    lmax = functools.reduce(jnp.maximum, maxs)
    denom = lmax - lmin + 1e-6
    for si in range(H // SLAB):
        sl = slice(si * SLAB, (si + 1) * SLAB)
        l255 = _dx(lsc[sl, :] - lmin, denom) * 255.0
        q = jnp.clip(jnp.round(l255), 0.0, 255.0)
        lq_ref[0, sl, :] = q
        lsc[sl, :] = q
    binrow = jax.lax.broadcasted_iota(jnp.int32, (1, _NBINS), 1).astype(
        jnp.float32)

    def tile_body(t, carry):
        ty = t // 4
        txp = t % 4
        sub2 = lsc[pl.ds(ty * TH, TH), pl.ds(txp * 128, 128)]
        accA = jnp.zeros((TH, _NBINS), jnp.float32)
        for c in range(TH):
            col = sub2[:, c:c + 1]
            accA = accA + jnp.where(col == binrow, 1.0, 0.0)
        hA = jnp.sum(accA, axis=0, keepdims=True)
        hist_sc[pl.ds(txp * 16 + ty, 1)] = hA.reshape(1, 1, _NBINS)
        accB = jnp.zeros((TH, _NBINS), jnp.float32)
        for c in range(TH):
            col = sub2[:, TH + c:TH + c + 1]
            accB = accB + jnp.where(col == binrow, 1.0, 0.0)
        hB = jnp.sum(accB, axis=0, keepdims=True)
        hist_sc[pl.ds(txp * 16 + 8 + ty, 1)] = hB.reshape(1, 1, _NBINS)
        return carry

    jax.lax.fori_loop(0, 32, tile_body, 0)
    hist = hist_sc[...].reshape(64, _NBINS)
    area = np.float32(TH * TH)
    clipped = jnp.minimum(hist, np.float32(32.0))
    excess = jnp.sum(hist - clipped, axis=1, keepdims=True)
    term = clipped + excess / np.float32(_NBINS)
    ii = jax.lax.broadcasted_iota(jnp.int32, (_NBINS, _NBINS), 0)
    jj = jax.lax.broadcasted_iota(jnp.int32, (_NBINS, _NBINS), 1)
    tri = jnp.where(ii <= jj, 1.0, 0.0)
    cum = jax.lax.dot(term, tri, precision=jax.lax.Precision.HIGHEST,
                      preferred_element_type=jnp.float32)
    luts_ref[0, 0:64, :] = jnp.round(cum * np.float32(255.0 / (TH * TH)))
    ghist = jnp.sum(hist, axis=0, keepdims=True)
    gcum = jax.lax.dot(ghist, tri, precision=jax.lax.Precision.HIGHEST,
                       preferred_element_type=jnp.float32)
    glut = jnp.round(gcum * np.float32(255.0 / (H * W)))
    luts_ref[0, 64:72, :] = jnp.broadcast_to(glut, (8, _NBINS))


def _lab_to_rgb_planes(l255, A, B):
    l = l255 * np.float32(100.0 / 255.0)
    fy = (l + 16.0) / 116.0
    fx = fy + A / 500.0
    fz = fy - B / 200.0

    def finv(f):
        f3 = f * f * f
        return jnp.where(f3 > _EPS, f3, (116.0 * f - 16.0) / _KAPPA)

    X = finv(fx) * 0.95047
    Y = finv(fy)
    Z = finv(fz) * 1.08883
    R = X * 3.2404542 + Y * (-1.5371385) + Z * (-0.4985314)
    G = X * (-0.9692660) + Y * 1.8760108 + Z * 0.0415560
    Bc = X * 0.0556434 + Y * (-0.2040259) + Z * 1.0572252

    def srgb(c):
        c = jnp.maximum(c, 0.0)
        p = 1.055 * jnp.exp2(jnp.log2(c) * np.float32(1.0 / 2.4)) - 0.055
        out = jnp.where(c <= 0.0031308, c * 12.92, p)
        return jnp.clip(out, 0.0, 1.0) * 255.0

    return srgb(R), srgb(G), srgb(Bc)


def _apply_kernel(lq_ref, a_ref, b_ref, luts_ref, eq_ref, cl_ref):
    ROWS = 32
    W = 512
    TH = 64
    v = lq_ref[0]
    vi = v.astype(jnp.int32)
    vlo = jnp.bitwise_and(vi, 127)
    hi_sel = vi >= 128
    si = pl.program_id(1)
    yrow = jax.lax.broadcasted_iota(jnp.int32, (ROWS, 1), 0) + si * ROWS
    cy = jnp.clip((yrow.astype(jnp.float32) + 0.5) / np.float32(TH) - 0.5,
                  0.0, np.float32(_GRID - 1))
    y0s = jnp.clip((si - 1) // 2, 0, _GRID - 2)
    wy = cy - y0s.astype(jnp.float32)
    xcol = jax.lax.broadcasted_iota(jnp.int32, (1, W), 1).astype(jnp.float32)
    cx = jnp.clip((xcol + 0.5) / np.float32(TH) - 0.5, 0.0,
                  np.float32(_GRID - 1))
    x0f = jnp.clip(jnp.floor(cx), 0.0, np.float32(_GRID - 2))
    wx = cx - x0f
    x0i = x0f.astype(jnp.int32)
    x1i = x0i + 1
    grow = luts_ref[0, 64:65, :]
    glo = jnp.broadcast_to(grow[:, 0:128], (ROWS, 128))
    ghi = jnp.broadcast_to(grow[:, 128:256], (ROWS, 128))
    eqv = jnp.where(hi_sel,
                    jnp.take_along_axis(ghi, vlo, axis=1),
                    jnp.take_along_axis(glo, vlo, axis=1))
    rmask0 = jax.lax.broadcasted_iota(jnp.int32, (8, _NBINS), 0) == y0s
    rmask1 = jax.lax.broadcasted_iota(jnp.int32, (8, _NBINS), 0) == (y0s + 1)
    v0 = jnp.zeros((ROWS, W), jnp.float32)
    v1 = jnp.zeros((ROWS, W), jnp.float32)
    for tx in range(_GRID):
        T8 = luts_ref[0, tx * 8:(tx + 1) * 8, :]
        t0 = jnp.sum(jnp.where(rmask0, T8, 0.0), axis=0, keepdims=True)
        t1 = jnp.sum(jnp.where(rmask1, T8, 0.0), axis=0, keepdims=True)
        lutY = t0 * (1.0 - wy) + t1 * wy
        gv = jnp.where(hi_sel,
                       jnp.take_along_axis(lutY[:, 128:256], vlo, axis=1),
                       jnp.take_along_axis(lutY[:, 0:128], vlo, axis=1))
        v0 = v0 + jnp.where(x0i == tx, gv, 0.0)
        v1 = v1 + jnp.where(x1i == tx, gv, 0.0)
    clv = v0 * (1.0 - wx) + v1 * wx
    A = a_ref[0]
    B = b_ref[0]
    r1, g1, b1 = _lab_to_rgb_planes(eqv, A, B)
    eq_ref[0, 0] = r1
    eq_ref[0, 1] = g1
    eq_ref[0, 2] = b1
    r2, g2, b2 = _lab_to_rgb_planes(clv, A, B)
    cl_ref[0, 0] = r2
    cl_ref[0, 1] = g2
    cl_ref[0, 2] = b2


def kernel(x):
    N, C, H, W = x.shape
    lq, a, b, luts = pl.pallas_call(
        _stats_kernel,
        grid=(N,),
        in_specs=[pl.BlockSpec((1, 3, H, W), lambda n: (n, 0, 0, 0))],
        out_specs=[
            pl.BlockSpec((1, H, W), lambda n: (n, 0, 0)),
            pl.BlockSpec((1, H, W), lambda n: (n, 0, 0)),
            pl.BlockSpec((1, H, W), lambda n: (n, 0, 0)),
            pl.BlockSpec((1, 72, _NBINS), lambda n: (n, 0, 0)),
        ],
        out_shape=[
            jax.ShapeDtypeStruct((N, H, W), jnp.float32),
            jax.ShapeDtypeStruct((N, H, W), jnp.float32),
            jax.ShapeDtypeStruct((N, H, W), jnp.float32),
            jax.ShapeDtypeStruct((N, 72, _NBINS), jnp.float32),
        ],
        scratch_shapes=[
            pltpu.VMEM((H, W), jnp.float32),
            pltpu.VMEM((64, 1, _NBINS), jnp.float32),
        ],
        compiler_params=pltpu.CompilerParams(
            dimension_semantics=("parallel",),
            vmem_limit_bytes=48 * 1024 * 1024,
        ),
    )(x)
    eq, cl = pl.pallas_call(
        _apply_kernel,
        grid=(N, H // 32),
        in_specs=[
            pl.BlockSpec((1, 32, W), lambda n, s: (n, s, 0)),
            pl.BlockSpec((1, 32, W), lambda n, s: (n, s, 0)),
            pl.BlockSpec((1, 32, W), lambda n, s: (n, s, 0)),
            pl.BlockSpec((1, 72, _NBINS), lambda n, s: (n, 0, 0)),
        ],
        out_specs=[
            pl.BlockSpec((1, 3, 32, W), lambda n, s: (n, 0, s, 0)),
            pl.BlockSpec((1, 3, 32, W), lambda n, s: (n, 0, s, 0)),
        ],
        out_shape=[
            jax.ShapeDtypeStruct((N, C, H, W), jnp.float32),
            jax.ShapeDtypeStruct((N, C, H, W), jnp.float32),
        ],
        compiler_params=pltpu.CompilerParams(
            dimension_semantics=("parallel", "arbitrary"),
            vmem_limit_bytes=48 * 1024 * 1024,
        ),
    )(lq, a, b, luts)
    return eq, cl


# speedup vs baseline: 1.2107x; 1.2107x over previous
import functools
import numpy as np
import jax
import jax.numpy as jnp
from jax.experimental import pallas as pl
from jax.experimental.pallas import tpu as pltpu

_EPS = np.float32(216.0 / 24389.0)
_KAPPA = np.float32(24389.0 / 27.0)
_NBINS = 256
_GRID = 8


def _dx(x, y):
    q = x / y
    return q + (x - q * y) / y


def _srgb_to_linear(c):
    u = _dx(c + 0.055, np.float32(1.055))
    p = jnp.exp(jnp.log(u) * np.float32(2.4))
    return jnp.where(c <= 0.04045, c / 12.92, p)


def _cbrt(t):
    return jnp.exp(jnp.log(t) * np.float32(1.0 / 3.0))


def _lab_f(t):
    return jnp.where(t > _EPS, _cbrt(t),
                     _dx(_KAPPA * t + 16.0, np.float32(116.0)))


def _stats_kernel(x_ref, lq_ref, a_ref, b_ref, luts_ref, lsc, hist_sc):
    H = W = 512
    TH = 64
    SLAB = 64
    mins = []
    maxs = []
    for si in range(H // SLAB):
        sl = slice(si * SLAB, (si + 1) * SLAB)
        r = x_ref[0, 0, sl, :]
        g = x_ref[0, 1, sl, :]
        bch = x_ref[0, 2, sl, :]
        lr = _srgb_to_linear(r)
        lg = _srgb_to_linear(g)
        lb = _srgb_to_linear(bch)
        xx = (lr * 0.4124564 + lg * 0.3575761 + lb * 0.1804375) / 0.95047
        yy = lr * 0.2126729 + lg * 0.7151522 + lb * 0.0721750
        zz = (lr * 0.0193339 + lg * 0.1191920 + lb * 0.9503041) / 1.08883
        fx = _lab_f(xx)
        fy = _lab_f(yy)
        fz = _lab_f(zz)
        L = 116.0 * fy - 16.0
        a_ref[0, sl, :] = 500.0 * (fx - fy)
        b_ref[0, sl, :] = 200.0 * (fy - fz)
        lsc[sl, :] = L
        mins.append(jnp.min(L, axis=(0, 1), keepdims=True))
        maxs.append(jnp.max(L, axis=(0, 1), keepdims=True))
    lmin = functools.reduce(jnp.minimum, mins)
    lmax = functools.reduce(jnp.maximum, maxs)
    denom = lmax - lmin + 1e-6
    for si in range(H // SLAB):
        sl = slice(si * SLAB, (si + 1) * SLAB)
        l255 = _dx(lsc[sl, :] - lmin, denom) * 255.0
        q = jnp.clip(jnp.round(l255), 0.0, 255.0)
        lq_ref[0, sl, :] = q
        lsc[sl, :] = q
    iota64 = jax.lax.broadcasted_iota(jnp.int32, (TH, 1), 0).astype(
        jnp.float32)
    lam = jax.lax.broadcasted_iota(jnp.int32, (8, W), 1)
    tt = jax.lax.broadcasted_iota(jnp.int32, (8, W), 0)
    tmask = jnp.where(jax.lax.shift_right_logical(lam, 6) == tt, 1.0, 0.0)
    cdims = (((1,), (1,)), ((), ()))

    def slab_body(t, carry):
        ty = t // 2
        cp = t % 2
        halves = []
        for ci in range(2):
            c0 = (cp * 2 + ci) * TH
            c0f = jnp.asarray(c0, jnp.float32)
            acc = jnp.zeros((TH, W), jnp.float32)
            for r8 in range(8):
                rows8 = lsc[pl.ds(ty * TH + r8 * 8, 8), :]
                for r in range(8):
                    row = rows8[r:r + 1, :]
                    acc = acc + jnp.where((row - c0f) == iota64, 1.0, 0.0)
            halves.append(jax.lax.dot_general(
                tmask, acc, cdims, precision=jax.lax.Precision.HIGHEST,
                preferred_element_type=jnp.float32))
        hist_sc[pl.ds(ty * 8, 8), pl.ds(cp * 128, 128)] = jnp.concatenate(
            halves, axis=1)
        return carry

    jax.lax.fori_loop(0, 16, slab_body, 0)
    hist = hist_sc[...]
    area = np.float32(TH * TH)
    clipped = jnp.minimum(hist, np.float32(32.0))
    excess = jnp.sum(hist - clipped, axis=1, keepdims=True)
    term = clipped + excess / np.float32(_NBINS)
    ii = jax.lax.broadcasted_iota(jnp.int32, (_NBINS, _NBINS), 0)
    jj = jax.lax.broadcasted_iota(jnp.int32, (_NBINS, _NBINS), 1)
    tri = jnp.where(ii <= jj, 1.0, 0.0)
    cum = jax.lax.dot(term, tri, precision=jax.lax.Precision.HIGHEST,
                      preferred_element_type=jnp.float32)
    luts_ref[0, 0:64, :] = jnp.round(cum * np.float32(255.0 / (TH * TH)))
    ghist = jnp.sum(hist, axis=0, keepdims=True)
    gcum = jax.lax.dot(ghist, tri, precision=jax.lax.Precision.HIGHEST,
                       preferred_element_type=jnp.float32)
    glut = jnp.round(gcum * np.float32(255.0 / (H * W)))
    luts_ref[0, 64:72, :] = jnp.broadcast_to(glut, (8, _NBINS))


def _lab_to_rgb_planes(l255, A, B):
    l = l255 * np.float32(100.0 / 255.0)
    fy = (l + 16.0) / 116.0
    fx = fy + A / 500.0
    fz = fy - B / 200.0

    def finv(f):
        f3 = f * f * f
        return jnp.where(f3 > _EPS, f3, (116.0 * f - 16.0) / _KAPPA)

    X = finv(fx) * 0.95047
    Y = finv(fy)
    Z = finv(fz) * 1.08883
    R = X * 3.2404542 + Y * (-1.5371385) + Z * (-0.4985314)
    G = X * (-0.9692660) + Y * 1.8760108 + Z * 0.0415560
    Bc = X * 0.0556434 + Y * (-0.2040259) + Z * 1.0572252

    def srgb(c):
        c = jnp.maximum(c, 0.0)
        p = 1.055 * jnp.exp2(jnp.log2(c) * np.float32(1.0 / 2.4)) - 0.055
        out = jnp.where(c <= 0.0031308, c * 12.92, p)
        return jnp.clip(out, 0.0, 1.0) * 255.0

    return srgb(R), srgb(G), srgb(Bc)


def _apply_kernel(lq_ref, a_ref, b_ref, luts_ref, eq_ref, cl_ref):
    ROWS = 32
    W = 512
    TH = 64
    v = lq_ref[0]
    vi = v.astype(jnp.int32)
    vlo = jnp.bitwise_and(vi, 127)
    hi_sel = vi >= 128
    si = pl.program_id(1)
    yrow = jax.lax.broadcasted_iota(jnp.int32, (ROWS, 1), 0) + si * ROWS
    cy = jnp.clip((yrow.astype(jnp.float32) + 0.5) / np.float32(TH) - 0.5,
                  0.0, np.float32(_GRID - 1))
    y0s = jnp.clip((si - 1) // 2, 0, _GRID - 2)
    wy = cy - y0s.astype(jnp.float32)
    xcol = jax.lax.broadcasted_iota(jnp.int32, (1, W), 1).astype(jnp.float32)
    cx = jnp.clip((xcol + 0.5) / np.float32(TH) - 0.5, 0.0,
                  np.float32(_GRID - 1))
    x0f = jnp.clip(jnp.floor(cx), 0.0, np.float32(_GRID - 2))
    wx = cx - x0f
    x0i = x0f.astype(jnp.int32)
    x1i = x0i + 1
    grow = luts_ref[0, 64:65, :]
    glo = jnp.broadcast_to(grow[:, 0:128], (ROWS, 128))
    ghi = jnp.broadcast_to(grow[:, 128:256], (ROWS, 128))
    eqv = jnp.where(hi_sel,
                    jnp.take_along_axis(ghi, vlo, axis=1),
                    jnp.take_along_axis(glo, vlo, axis=1))
    T8y0 = luts_ref[0, pl.ds(y0s * 8, 8), :]
    T8y1 = luts_ref[0, pl.ds((y0s + 1) * 8, 8), :]
    v0 = jnp.zeros((ROWS, W), jnp.float32)
    v1 = jnp.zeros((ROWS, W), jnp.float32)
    for tx in range(_GRID):
        t0 = T8y0[tx:tx + 1, :]
        t1 = T8y1[tx:tx + 1, :]
        lutY = t0 * (1.0 - wy) + t1 * wy
        gv = jnp.where(hi_sel,
                       jnp.take_along_axis(lutY[:, 128:256], vlo, axis=1),
                       jnp.take_along_axis(lutY[:, 0:128], vlo, axis=1))
        v0 = v0 + jnp.where(x0i == tx, gv, 0.0)
        v1 = v1 + jnp.where(x1i == tx, gv, 0.0)
    clv = v0 * (1.0 - wx) + v1 * wx
    A = a_ref[0]
    B = b_ref[0]
    r1, g1, b1 = _lab_to_rgb_planes(eqv, A, B)
    eq_ref[0, 0] = r1
    eq_ref[0, 1] = g1
    eq_ref[0, 2] = b1
    r2, g2, b2 = _lab_to_rgb_planes(clv, A, B)
    cl_ref[0, 0] = r2
    cl_ref[0, 1] = g2
    cl_ref[0, 2] = b2


def kernel(x):
    N, C, H, W = x.shape
    lq, a, b, luts = pl.pallas_call(
        _stats_kernel,
        grid=(N,),
        in_specs=[pl.BlockSpec((1, 3, H, W), lambda n: (n, 0, 0, 0))],
        out_specs=[
            pl.BlockSpec((1, H, W), lambda n: (n, 0, 0)),
            pl.BlockSpec((1, H, W), lambda n: (n, 0, 0)),
            pl.BlockSpec((1, H, W), lambda n: (n, 0, 0)),
            pl.BlockSpec((1, 72, _NBINS), lambda n: (n, 0, 0)),
        ],
        out_shape=[
            jax.ShapeDtypeStruct((N, H, W), jnp.float32),
            jax.ShapeDtypeStruct((N, H, W), jnp.float32),
            jax.ShapeDtypeStruct((N, H, W), jnp.float32),
            jax.ShapeDtypeStruct((N, 72, _NBINS), jnp.float32),
        ],
        scratch_shapes=[
            pltpu.VMEM((H, W), jnp.float32),
            pltpu.VMEM((64, _NBINS), jnp.float32),
        ],
        compiler_params=pltpu.CompilerParams(
            dimension_semantics=("parallel",),
            vmem_limit_bytes=48 * 1024 * 1024,
        ),
    )(x)
    eq, cl = pl.pallas_call(
        _apply_kernel,
        grid=(N, H // 32),
        in_specs=[
            pl.BlockSpec((1, 32, W), lambda n, s: (n, s, 0)),
            pl.BlockSpec((1, 32, W), lambda n, s: (n, s, 0)),
            pl.BlockSpec((1, 32, W), lambda n, s: (n, s, 0)),
            pl.BlockSpec((1, 72, _NBINS), lambda n, s: (n, 0, 0)),
        ],
        out_specs=[
            pl.BlockSpec((1, 3, 32, W), lambda n, s: (n, 0, s, 0)),
            pl.BlockSpec((1, 3, 32, W), lambda n, s: (n, 0, s, 0)),
        ],
        out_shape=[
            jax.ShapeDtypeStruct((N, C, H, W), jnp.float32),
            jax.ShapeDtypeStruct((N, C, H, W), jnp.float32),
        ],
        compiler_params=pltpu.CompilerParams(
            dimension_semantics=("parallel", "arbitrary"),
            vmem_limit_bytes=48 * 1024 * 1024,
        ),
    )(lq, a, b, luts)
    return eq, cl


# speedup vs baseline: 1.6661x; 1.3762x over previous
import functools
import numpy as np
import jax
import jax.numpy as jnp
from jax.experimental import pallas as pl
from jax.experimental.pallas import tpu as pltpu

_EPS = np.float32(216.0 / 24389.0)
_KAPPA = np.float32(24389.0 / 27.0)
_NBINS = 256
_GRID = 8


def _dx(x, y):
    q = x / y
    return q + (x - q * y) / y


def _srgb_to_linear(c):
    u = _dx(c + 0.055, np.float32(1.055))
    p = jnp.exp(jnp.log(u) * np.float32(2.4))
    return jnp.where(c <= 0.04045, c / 12.92, p)


def _cbrt(t):
    return jnp.exp(jnp.log(t) * np.float32(1.0 / 3.0))


def _lab_f(t):
    return jnp.where(t > _EPS, _cbrt(t),
                     _dx(_KAPPA * t + 16.0, np.float32(116.0)))


def _stats_kernel(x_ref, lq_ref, a_ref, b_ref, luts_ref, lsc, hist_sc):
    H = W = 512
    TH = 64
    SLAB = 64
    mins = []
    maxs = []
    for si in range(H // SLAB):
        sl = slice(si * SLAB, (si + 1) * SLAB)
        r = x_ref[0, 0, sl, :]
        g = x_ref[0, 1, sl, :]
        bch = x_ref[0, 2, sl, :]
        lr = _srgb_to_linear(r)
        lg = _srgb_to_linear(g)
        lb = _srgb_to_linear(bch)
        xx = (lr * 0.4124564 + lg * 0.3575761 + lb * 0.1804375) / 0.95047
        yy = lr * 0.2126729 + lg * 0.7151522 + lb * 0.0721750
        zz = (lr * 0.0193339 + lg * 0.1191920 + lb * 0.9503041) / 1.08883
        fx = _lab_f(xx)
        fy = _lab_f(yy)
        fz = _lab_f(zz)
        L = 116.0 * fy - 16.0
        a_ref[0, sl, :] = 500.0 * (fx - fy)
        b_ref[0, sl, :] = 200.0 * (fy - fz)
        lsc[sl, :] = L
        mins.append(jnp.min(L, axis=(0, 1), keepdims=True))
        maxs.append(jnp.max(L, axis=(0, 1), keepdims=True))
    lmin = functools.reduce(jnp.minimum, mins)
    lmax = functools.reduce(jnp.maximum, maxs)
    denom = lmax - lmin + 1e-6
    for si in range(H // SLAB):
        sl = slice(si * SLAB, (si + 1) * SLAB)
        l255 = _dx(lsc[sl, :] - lmin, denom) * 255.0
        q = jnp.clip(jnp.round(l255), 0.0, 255.0)
        lq_ref[0, sl, :] = q
        lsc[sl, :] = q
    one_bf = jnp.bfloat16(1.0)
    zero_bf = jnp.bfloat16(0.0)
    iota128 = jax.lax.broadcasted_iota(jnp.int32, (128, 1), 0).astype(
        jnp.bfloat16)
    lam = jax.lax.broadcasted_iota(jnp.int32, (8, W), 1)
    tt = jax.lax.broadcasted_iota(jnp.int32, (8, W), 0)
    tmask = jnp.where(jax.lax.shift_right_logical(lam, 6) == tt, 1.0,
                      0.0).astype(jnp.bfloat16)
    cdims = (((1,), (1,)), ((), ()))

    def slab_body(t, carry):
        ty = t // 2
        cp = t % 2
        c0f = (cp * 128).astype(jnp.bfloat16)
        acc = jnp.zeros((128, W), jnp.bfloat16)
        for r8 in range(8):
            rows8 = lsc[pl.ds(ty * TH + r8 * 8, 8), :]
            for r in range(8):
                row = rows8[r:r + 1, :].astype(jnp.bfloat16)
                acc = acc + jnp.where((row - c0f) == iota128, one_bf, zero_bf)
        h8 = jax.lax.dot_general(tmask, acc, cdims,
                                 preferred_element_type=jnp.float32)
        hist_sc[pl.ds(ty * 8, 8), pl.ds(cp * 128, 128)] = h8
        return carry

    jax.lax.fori_loop(0, 16, slab_body, 0)
    hist = hist_sc[...]
    area = np.float32(TH * TH)
    clipped = jnp.minimum(hist, np.float32(32.0))
    excess = jnp.sum(hist - clipped, axis=1, keepdims=True)
    term = clipped + excess / np.float32(_NBINS)
    ii = jax.lax.broadcasted_iota(jnp.int32, (_NBINS, _NBINS), 0)
    jj = jax.lax.broadcasted_iota(jnp.int32, (_NBINS, _NBINS), 1)
    tri = jnp.where(ii <= jj, 1.0, 0.0)
    cum = jax.lax.dot(term, tri, precision=jax.lax.Precision.HIGHEST,
                      preferred_element_type=jnp.float32)
    luts_ref[0, 0:64, :] = jnp.round(cum * np.float32(255.0 / (TH * TH)))
    ghist = jnp.sum(hist, axis=0, keepdims=True)
    gcum = jax.lax.dot(ghist, tri, precision=jax.lax.Precision.HIGHEST,
                       preferred_element_type=jnp.float32)
    glut = jnp.round(gcum * np.float32(255.0 / (H * W)))
    luts_ref[0, 64:72, :] = jnp.broadcast_to(glut, (8, _NBINS))


def _lab_to_rgb_planes(l255, A, B):
    l = l255 * np.float32(100.0 / 255.0)
    fy = (l + 16.0) / 116.0
    fx = fy + A / 500.0
    fz = fy - B / 200.0

    def finv(f):
        f3 = f * f * f
        return jnp.where(f3 > _EPS, f3, (116.0 * f - 16.0) / _KAPPA)

    X = finv(fx) * 0.95047
    Y = finv(fy)
    Z = finv(fz) * 1.08883
    R = X * 3.2404542 + Y * (-1.5371385) + Z * (-0.4985314)
    G = X * (-0.9692660) + Y * 1.8760108 + Z * 0.0415560
    Bc = X * 0.0556434 + Y * (-0.2040259) + Z * 1.0572252

    def srgb(c):
        c = jnp.maximum(c, 0.0)
        p = 1.055 * jnp.exp2(jnp.log2(c) * np.float32(1.0 / 2.4)) - 0.055
        out = jnp.where(c <= 0.0031308, c * 12.92, p)
        return jnp.clip(out, 0.0, 1.0) * 255.0

    return srgb(R), srgb(G), srgb(Bc)


def _apply_kernel(lq_ref, a_ref, b_ref, luts_ref, eq_ref, cl_ref):
    ROWS = 32
    W = 512
    TH = 64
    v = lq_ref[0]
    vi = v.astype(jnp.int32)
    vlo = jnp.bitwise_and(vi, 127)
    hi_sel = vi >= 128
    si = pl.program_id(1)
    yrow = jax.lax.broadcasted_iota(jnp.int32, (ROWS, 1), 0) + si * ROWS
    cy = jnp.clip((yrow.astype(jnp.float32) + 0.5) / np.float32(TH) - 0.5,
                  0.0, np.float32(_GRID - 1))
    y0s = jnp.clip((si - 1) // 2, 0, _GRID - 2)
    wy = cy - y0s.astype(jnp.float32)
    xcol = jax.lax.broadcasted_iota(jnp.int32, (1, W), 1).astype(jnp.float32)
    cx = jnp.clip((xcol + 0.5) / np.float32(TH) - 0.5, 0.0,
                  np.float32(_GRID - 1))
    x0f = jnp.clip(jnp.floor(cx), 0.0, np.float32(_GRID - 2))
    wx = cx - x0f
    x0i = x0f.astype(jnp.int32)
    x1i = x0i + 1
    grow = luts_ref[0, 64:65, :]
    glo = jnp.broadcast_to(grow[:, 0:128], (ROWS, 128))
    ghi = jnp.broadcast_to(grow[:, 128:256], (ROWS, 128))
    eqv = jnp.where(hi_sel,
                    jnp.take_along_axis(ghi, vlo, axis=1),
                    jnp.take_along_axis(glo, vlo, axis=1))
    T8y0 = luts_ref[0, pl.ds(y0s * 8, 8), :]
    T8y1 = luts_ref[0, pl.ds((y0s + 1) * 8, 8), :]
    v0 = jnp.zeros((ROWS, W), jnp.float32)
    v1 = jnp.zeros((ROWS, W), jnp.float32)
    for tx in range(_GRID):
        t0 = T8y0[tx:tx + 1, :]
        t1 = T8y1[tx:tx + 1, :]
        lutY = t0 * (1.0 - wy) + t1 * wy
        gv = jnp.where(hi_sel,
                       jnp.take_along_axis(lutY[:, 128:256], vlo, axis=1),
                       jnp.take_along_axis(lutY[:, 0:128], vlo, axis=1))
        v0 = v0 + jnp.where(x0i == tx, gv, 0.0)
        v1 = v1 + jnp.where(x1i == tx, gv, 0.0)
    clv = v0 * (1.0 - wx) + v1 * wx
    A = a_ref[0]
    B = b_ref[0]
    r1, g1, b1 = _lab_to_rgb_planes(eqv, A, B)
    eq_ref[0, 0] = r1
    eq_ref[0, 1] = g1
    eq_ref[0, 2] = b1
    r2, g2, b2 = _lab_to_rgb_planes(clv, A, B)
    cl_ref[0, 0] = r2
    cl_ref[0, 1] = g2
    cl_ref[0, 2] = b2


def kernel(x):
    N, C, H, W = x.shape
    lq, a, b, luts = pl.pallas_call(
        _stats_kernel,
        grid=(N,),
        in_specs=[pl.BlockSpec((1, 3, H, W), lambda n: (n, 0, 0, 0))],
        out_specs=[
            pl.BlockSpec((1, H, W), lambda n: (n, 0, 0)),
            pl.BlockSpec((1, H, W), lambda n: (n, 0, 0)),
            pl.BlockSpec((1, H, W), lambda n: (n, 0, 0)),
            pl.BlockSpec((1, 72, _NBINS), lambda n: (n, 0, 0)),
        ],
        out_shape=[
            jax.ShapeDtypeStruct((N, H, W), jnp.float32),
            jax.ShapeDtypeStruct((N, H, W), jnp.float32),
            jax.ShapeDtypeStruct((N, H, W), jnp.float32),
            jax.ShapeDtypeStruct((N, 72, _NBINS), jnp.float32),
        ],
        scratch_shapes=[
            pltpu.VMEM((H, W), jnp.float32),
            pltpu.VMEM((64, _NBINS), jnp.float32),
        ],
        compiler_params=pltpu.CompilerParams(
            dimension_semantics=("parallel",),
            vmem_limit_bytes=48 * 1024 * 1024,
        ),
    )(x)
    eq, cl = pl.pallas_call(
        _apply_kernel,
        grid=(N, H // 32),
        in_specs=[
            pl.BlockSpec((1, 32, W), lambda n, s: (n, s, 0)),
            pl.BlockSpec((1, 32, W), lambda n, s: (n, s, 0)),
            pl.BlockSpec((1, 32, W), lambda n, s: (n, s, 0)),
            pl.BlockSpec((1, 72, _NBINS), lambda n, s: (n, 0, 0)),
        ],
        out_specs=[
            pl.BlockSpec((1, 3, 32, W), lambda n, s: (n, 0, s, 0)),
            pl.BlockSpec((1, 3, 32, W), lambda n, s: (n, 0, s, 0)),
        ],
        out_shape=[
            jax.ShapeDtypeStruct((N, C, H, W), jnp.float32),
            jax.ShapeDtypeStruct((N, C, H, W), jnp.float32),
        ],
        compiler_params=pltpu.CompilerParams(
            dimension_semantics=("parallel", "arbitrary"),
            vmem_limit_bytes=48 * 1024 * 1024,
        ),
    )(lq, a, b, luts)
    return eq, cl


# speedup vs baseline: 1.7092x; 1.0258x over previous
import functools
import numpy as np
import jax
import jax.numpy as jnp
from jax.experimental import pallas as pl
from jax.experimental.pallas import tpu as pltpu

_EPS = np.float32(216.0 / 24389.0)
_KAPPA = np.float32(24389.0 / 27.0)
_NBINS = 256
_GRID = 8


def _dx(x, y):
    q = x / y
    return q + (x - q * y) / y


def _srgb_to_linear(c):
    u = _dx(c + 0.055, np.float32(1.055))
    p = jnp.exp(jnp.log(u) * np.float32(2.4))
    return jnp.where(c <= 0.04045, c / 12.92, p)


def _cbrt(t):
    return jnp.exp(jnp.log(t) * np.float32(1.0 / 3.0))


def _lab_f(t):
    return jnp.where(t > _EPS, _cbrt(t),
                     _dx(_KAPPA * t + 16.0, np.float32(116.0)))


def _lab_to_rgb_planes(l255, A, B):
    l = l255 * np.float32(100.0 / 255.0)
    fy = (l + 16.0) / 116.0
    fx = fy + A / 500.0
    fz = fy - B / 200.0

    def finv(f):
        f3 = f * f * f
        return jnp.where(f3 > _EPS, f3, (116.0 * f - 16.0) / _KAPPA)

    X = finv(fx) * 0.95047
    Y = finv(fy)
    Z = finv(fz) * 1.08883
    R = X * 3.2404542 + Y * (-1.5371385) + Z * (-0.4985314)
    G = X * (-0.9692660) + Y * 1.8760108 + Z * 0.0415560
    Bc = X * 0.0556434 + Y * (-0.2040259) + Z * 1.0572252

    def srgb(c):
        c = jnp.maximum(c, 0.0)
        p = 1.055 * jnp.exp2(jnp.log2(c) * np.float32(1.0 / 2.4)) - 0.055
        out = jnp.where(c <= 0.0031308, c * 12.92, p)
        return jnp.clip(out, 0.0, 1.0) * 255.0

    return srgb(R), srgb(G), srgb(Bc)


def _wb_clahe_kernel(x_ref, eq_ref, cl_ref, lsc, a_sc, b_sc, hist_sc,
                     luts_sc):
    H = W = 512
    TH = 64
    SLAB = 64
    mins = []
    maxs = []
    for si in range(H // SLAB):
        sl = slice(si * SLAB, (si + 1) * SLAB)
        r = x_ref[0, 0, sl, :]
        g = x_ref[0, 1, sl, :]
        bch = x_ref[0, 2, sl, :]
        lr = _srgb_to_linear(r)
        lg = _srgb_to_linear(g)
        lb = _srgb_to_linear(bch)
        xx = (lr * 0.4124564 + lg * 0.3575761 + lb * 0.1804375) / 0.95047
        yy = lr * 0.2126729 + lg * 0.7151522 + lb * 0.0721750
        zz = (lr * 0.0193339 + lg * 0.1191920 + lb * 0.9503041) / 1.08883
        fx = _lab_f(xx)
        fy = _lab_f(yy)
        fz = _lab_f(zz)
        L = 116.0 * fy - 16.0
        a_sc[sl, :] = 500.0 * (fx - fy)
        b_sc[sl, :] = 200.0 * (fy - fz)
        lsc[sl, :] = L
        mins.append(jnp.min(L, axis=(0, 1), keepdims=True))
        maxs.append(jnp.max(L, axis=(0, 1), keepdims=True))
    lmin = functools.reduce(jnp.minimum, mins)
    lmax = functools.reduce(jnp.maximum, maxs)
    denom = lmax - lmin + 1e-6
    for si in range(H // SLAB):
        sl = slice(si * SLAB, (si + 1) * SLAB)
        l255 = _dx(lsc[sl, :] - lmin, denom) * 255.0
        lsc[sl, :] = jnp.clip(jnp.round(l255), 0.0, 255.0)
    one_bf = jnp.bfloat16(1.0)
    zero_bf = jnp.bfloat16(0.0)
    iota128 = jax.lax.broadcasted_iota(jnp.int32, (128, 1), 0).astype(
        jnp.bfloat16)
    lam = jax.lax.broadcasted_iota(jnp.int32, (8, W), 1)
    tt = jax.lax.broadcasted_iota(jnp.int32, (8, W), 0)
    tmask = jnp.where(jax.lax.shift_right_logical(lam, 6) == tt, 1.0,
                      0.0).astype(jnp.bfloat16)
    cdims = (((1,), (1,)), ((), ()))

    def slab_body(t, carry):
        ty = t // 2
        cp = t % 2
        c0f = (cp * 128).astype(jnp.bfloat16)
        acc = jnp.zeros((128, W), jnp.bfloat16)
        for r8 in range(8):
            rows8 = lsc[pl.ds(ty * TH + r8 * 8, 8), :]
            for r in range(8):
                row = rows8[r:r + 1, :].astype(jnp.bfloat16)
                acc = acc + jnp.where((row - c0f) == iota128, one_bf, zero_bf)
        h8 = jax.lax.dot_general(tmask, acc, cdims,
                                 preferred_element_type=jnp.float32)
        hist_sc[pl.ds(ty * 8, 8), pl.ds(cp * 128, 128)] = h8
        return carry

    jax.lax.fori_loop(0, 16, slab_body, 0)
    hist = hist_sc[...]
    clipped = jnp.minimum(hist, np.float32(32.0))
    excess = jnp.sum(hist - clipped, axis=1, keepdims=True)
    term = clipped + excess / np.float32(_NBINS)
    ii = jax.lax.broadcasted_iota(jnp.int32, (_NBINS, _NBINS), 0)
    jj = jax.lax.broadcasted_iota(jnp.int32, (_NBINS, _NBINS), 1)
    tri = jnp.where(ii <= jj, 1.0, 0.0)
    cum = jax.lax.dot(term, tri, precision=jax.lax.Precision.HIGHEST,
                      preferred_element_type=jnp.float32)
    luts_sc[0:64, :] = jnp.round(cum * np.float32(255.0 / (TH * TH)))
    ghist = jnp.sum(hist, axis=0, keepdims=True)
    gcum = jax.lax.dot(ghist, tri, precision=jax.lax.Precision.HIGHEST,
                       preferred_element_type=jnp.float32)
    glut = jnp.round(gcum * np.float32(255.0 / (H * W)))
    luts_sc[64:72, :] = jnp.broadcast_to(glut, (8, _NBINS))
    ROWS = 32
    xcol = jax.lax.broadcasted_iota(jnp.int32, (1, W), 1).astype(jnp.float32)
    cx = jnp.clip((xcol + 0.5) / np.float32(TH) - 0.5, 0.0,
                  np.float32(_GRID - 1))
    x0f = jnp.clip(jnp.floor(cx), 0.0, np.float32(_GRID - 2))
    wx = cx - x0f
    x0i = x0f.astype(jnp.int32)
    x1i = x0i + 1
    yiota = jax.lax.broadcasted_iota(jnp.int32, (ROWS, 1), 0)

    def apply_body(si, carry):
        sl = pl.ds(si * ROWS, ROWS)
        v = lsc[sl, :]
        vi = v.astype(jnp.int32)
        vlo = jnp.bitwise_and(vi, 127)
        hi_sel = vi >= 128
        yrow = yiota + si * ROWS
        cy = jnp.clip((yrow.astype(jnp.float32) + 0.5) / np.float32(TH) - 0.5,
                      0.0, np.float32(_GRID - 1))
        y0s = jnp.clip((si - 1) // 2, 0, _GRID - 2)
        wy = cy - y0s.astype(jnp.float32)
        grow = luts_sc[64:65, :]
        glo = jnp.broadcast_to(grow[:, 0:128], (ROWS, 128))
        ghi = jnp.broadcast_to(grow[:, 128:256], (ROWS, 128))
        eqv = jnp.where(hi_sel,
                        jnp.take_along_axis(ghi, vlo, axis=1),
                        jnp.take_along_axis(glo, vlo, axis=1))
        T8y0 = luts_sc[pl.ds(y0s * 8, 8), :]
        T8y1 = luts_sc[pl.ds((y0s + 1) * 8, 8), :]
        v0 = jnp.zeros((ROWS, W), jnp.float32)
        v1 = jnp.zeros((ROWS, W), jnp.float32)
        for tx in range(_GRID):
            t0 = T8y0[tx:tx + 1, :]
            t1 = T8y1[tx:tx + 1, :]
            lutY = t0 * (1.0 - wy) + t1 * wy
            gv = jnp.where(hi_sel,
                           jnp.take_along_axis(lutY[:, 128:256], vlo, axis=1),
                           jnp.take_along_axis(lutY[:, 0:128], vlo, axis=1))
            v0 = v0 + jnp.where(x0i == tx, gv, 0.0)
            v1 = v1 + jnp.where(x1i == tx, gv, 0.0)
        clv = v0 * (1.0 - wx) + v1 * wx
        A = a_sc[sl, :]
        B = b_sc[sl, :]
        r1, g1, b1 = _lab_to_rgb_planes(eqv, A, B)
        eq_ref[0, 0, sl, :] = r1
        eq_ref[0, 1, sl, :] = g1
        eq_ref[0, 2, sl, :] = b1
        r2, g2, b2 = _lab_to_rgb_planes(clv, A, B)
        cl_ref[0, 0, sl, :] = r2
        cl_ref[0, 1, sl, :] = g2
        cl_ref[0, 2, sl, :] = b2
        return carry

    jax.lax.fori_loop(0, H // ROWS, apply_body, 0)


def kernel(x):
    N, C, H, W = x.shape
    eq, cl = pl.pallas_call(
        _wb_clahe_kernel,
        grid=(N,),
        in_specs=[pl.BlockSpec((1, 3, H, W), lambda n: (n, 0, 0, 0))],
        out_specs=[
            pl.BlockSpec((1, 3, H, W), lambda n: (n, 0, 0, 0)),
            pl.BlockSpec((1, 3, H, W), lambda n: (n, 0, 0, 0)),
        ],
        out_shape=[
            jax.ShapeDtypeStruct((N, C, H, W), jnp.float32),
            jax.ShapeDtypeStruct((N, C, H, W), jnp.float32),
        ],
        scratch_shapes=[
            pltpu.VMEM((H, W), jnp.float32),
            pltpu.VMEM((H, W), jnp.float32),
            pltpu.VMEM((H, W), jnp.float32),
            pltpu.VMEM((64, _NBINS), jnp.float32),
            pltpu.VMEM((72, _NBINS), jnp.float32),
        ],
        compiler_params=pltpu.CompilerParams(
            dimension_semantics=("parallel",),
            vmem_limit_bytes=56 * 1024 * 1024,
        ),
    )(x)
    return eq, cl


# speedup vs baseline: 1.7097x; 1.0003x over previous
import functools
import numpy as np
import jax
import jax.numpy as jnp
from jax.experimental import pallas as pl
from jax.experimental.pallas import tpu as pltpu

_EPS = np.float32(216.0 / 24389.0)
_KAPPA = np.float32(24389.0 / 27.0)
_NBINS = 256
_GRID = 8


def _dx(x, y):
    q = x / y
    return q + (x - q * y) / y


def _srgb_to_linear(c):
    u = _dx(c + 0.055, np.float32(1.055))
    p = jnp.exp(jnp.log(u) * np.float32(2.4))
    return jnp.where(c <= 0.04045, c / 12.92, p)


def _cbrt(t):
    return jnp.exp(jnp.log(t) * np.float32(1.0 / 3.0))


def _lab_f(t):
    return jnp.where(t > _EPS, _cbrt(t),
                     _dx(_KAPPA * t + 16.0, np.float32(116.0)))


def _lab_to_rgb_planes(l255, A, B):
    l = l255 * np.float32(100.0 / 255.0)
    fy = (l + 16.0) / 116.0
    fx = fy + A / 500.0
    fz = fy - B / 200.0

    def finv(f):
        f3 = f * f * f
        return jnp.where(f3 > _EPS, f3, (116.0 * f - 16.0) / _KAPPA)

    X = finv(fx) * 0.95047
    Y = finv(fy)
    Z = finv(fz) * 1.08883
    R = X * 3.2404542 + Y * (-1.5371385) + Z * (-0.4985314)
    G = X * (-0.9692660) + Y * 1.8760108 + Z * 0.0415560
    Bc = X * 0.0556434 + Y * (-0.2040259) + Z * 1.0572252

    def srgb(c):
        c = jnp.maximum(c, 0.0)
        p = 1.055 * jnp.exp2(jnp.log2(c) * np.float32(1.0 / 2.4)) - 0.055
        out = jnp.where(c <= 0.0031308, c * 12.92, p)
        return jnp.clip(out, 0.0, 1.0) * 255.0

    return srgb(R), srgb(G), srgb(Bc)


def _wb_clahe_kernel(x_ref, eq_ref, cl_ref, lsc, a_sc, b_sc, hist_sc,
                     luts_sc):
    H = W = 512
    TH = 64
    SLAB = 64
    mins = []
    maxs = []
    for si in range(H // SLAB):
        sl = slice(si * SLAB, (si + 1) * SLAB)
        r = x_ref[0, 0, sl, :]
        g = x_ref[0, 1, sl, :]
        bch = x_ref[0, 2, sl, :]
        lr = _srgb_to_linear(r)
        lg = _srgb_to_linear(g)
        lb = _srgb_to_linear(bch)
        xx = (lr * 0.4124564 + lg * 0.3575761 + lb * 0.1804375) / 0.95047
        yy = lr * 0.2126729 + lg * 0.7151522 + lb * 0.0721750
        zz = (lr * 0.0193339 + lg * 0.1191920 + lb * 0.9503041) / 1.08883
        fx = _lab_f(xx)
        fy = _lab_f(yy)
        fz = _lab_f(zz)
        L = 116.0 * fy - 16.0
        a_sc[sl, :] = 500.0 * (fx - fy)
        b_sc[sl, :] = 200.0 * (fy - fz)
        lsc[sl, :] = L
        mins.append(jnp.min(L, axis=(0, 1), keepdims=True))
        maxs.append(jnp.max(L, axis=(0, 1), keepdims=True))
    lmin = functools.reduce(jnp.minimum, mins)
    lmax = functools.reduce(jnp.maximum, maxs)
    denom = lmax - lmin + 1e-6
    for si in range(H // SLAB):
        sl = slice(si * SLAB, (si + 1) * SLAB)
        l255 = _dx(lsc[sl, :] - lmin, denom) * 255.0
        lsc[sl, :] = jnp.clip(jnp.round(l255), 0.0, 255.0)
    one_bf = jnp.bfloat16(1.0)
    zero_bf = jnp.bfloat16(0.0)
    iota128 = jax.lax.broadcasted_iota(jnp.int32, (128, 1), 0).astype(
        jnp.bfloat16)
    lam = jax.lax.broadcasted_iota(jnp.int32, (8, W), 1)
    tt = jax.lax.broadcasted_iota(jnp.int32, (8, W), 0)
    tmask = jnp.where(jax.lax.shift_right_logical(lam, 6) == tt, 1.0,
                      0.0).astype(jnp.bfloat16)
    cdims = (((1,), (1,)), ((), ()))

    def slab_body(t, carry):
        ty = t // 2
        cp = t % 2
        c0f = (cp * 128).astype(jnp.bfloat16)
        acc = jnp.zeros((128, W), jnp.bfloat16)
        for r8 in range(8):
            rows8 = lsc[pl.ds(ty * TH + r8 * 8, 8), :].astype(jnp.bfloat16)
            for r in range(8):
                row = rows8[r:r + 1, :]
                acc = acc + jnp.where((row - c0f) == iota128, one_bf, zero_bf)
        h8 = jax.lax.dot_general(tmask, acc, cdims,
                                 preferred_element_type=jnp.float32)
        hist_sc[pl.ds(ty * 8, 8), pl.ds(cp * 128, 128)] = h8
        return carry

    jax.lax.fori_loop(0, 16, slab_body, 0)
    hist = hist_sc[...]
    clipped = jnp.minimum(hist, np.float32(32.0))
    excess = jnp.sum(hist - clipped, axis=1, keepdims=True)
    term = clipped + excess / np.float32(_NBINS)
    ii = jax.lax.broadcasted_iota(jnp.int32, (_NBINS, _NBINS), 0)
    jj = jax.lax.broadcasted_iota(jnp.int32, (_NBINS, _NBINS), 1)
    tri = jnp.where(ii <= jj, 1.0, 0.0)
    cum = jax.lax.dot(term, tri, precision=jax.lax.Precision.HIGHEST,
                      preferred_element_type=jnp.float32)
    luts_sc[0:64, :] = jnp.round(cum * np.float32(255.0 / (TH * TH)))
    ghist = jnp.sum(hist, axis=0, keepdims=True)
    gcum = jax.lax.dot(ghist, tri, precision=jax.lax.Precision.HIGHEST,
                       preferred_element_type=jnp.float32)
    glut = jnp.round(gcum * np.float32(255.0 / (H * W)))
    luts_sc[64:72, :] = jnp.broadcast_to(glut, (8, _NBINS))
    ROWS = 32
    xcol = jax.lax.broadcasted_iota(jnp.int32, (1, W), 1).astype(jnp.float32)
    cx = jnp.clip((xcol + 0.5) / np.float32(TH) - 0.5, 0.0,
                  np.float32(_GRID - 1))
    x0f = jnp.clip(jnp.floor(cx), 0.0, np.float32(_GRID - 2))
    wx = cx - x0f
    x0i = x0f.astype(jnp.int32)
    x1i = x0i + 1
    yiota = jax.lax.broadcasted_iota(jnp.int32, (ROWS, 1), 0)

    def apply_body(si, carry):
        sl = pl.ds(si * ROWS, ROWS)
        v = lsc[sl, :]
        vi = v.astype(jnp.int32)
        vlo = jnp.bitwise_and(vi, 127)
        hi_sel = vi >= 128
        yrow = yiota + si * ROWS
        cy = jnp.clip((yrow.astype(jnp.float32) + 0.5) / np.float32(TH) - 0.5,
                      0.0, np.float32(_GRID - 1))
        y0s = jnp.clip((si - 1) // 2, 0, _GRID - 2)
        wy = cy - y0s.astype(jnp.float32)
        grow = luts_sc[64:65, :]
        glo = jnp.broadcast_to(grow[:, 0:128], (ROWS, 128))
        ghi = jnp.broadcast_to(grow[:, 128:256], (ROWS, 128))
        eqv = jnp.where(hi_sel,
                        jnp.take_along_axis(ghi, vlo, axis=1),
                        jnp.take_along_axis(glo, vlo, axis=1))
        T8y0 = luts_sc[pl.ds(y0s * 8, 8), :]
        T8y1 = luts_sc[pl.ds((y0s + 1) * 8, 8), :]
        v0 = jnp.zeros((ROWS, W), jnp.float32)
        v1 = jnp.zeros((ROWS, W), jnp.float32)
        for tx in range(_GRID):
            t0 = T8y0[tx:tx + 1, :]
            t1 = T8y1[tx:tx + 1, :]
            lutY = t0 * (1.0 - wy) + t1 * wy
            gv = jnp.where(hi_sel,
                           jnp.take_along_axis(lutY[:, 128:256], vlo, axis=1),
                           jnp.take_along_axis(lutY[:, 0:128], vlo, axis=1))
            v0 = v0 + jnp.where(x0i == tx, gv, 0.0)
            v1 = v1 + jnp.where(x1i == tx, gv, 0.0)
        clv = v0 * (1.0 - wx) + v1 * wx
        A = a_sc[sl, :]
        B = b_sc[sl, :]
        r1, g1, b1 = _lab_to_rgb_planes(eqv, A, B)
        eq_ref[0, 0, sl, :] = r1
        eq_ref[0, 1, sl, :] = g1
        eq_ref[0, 2, sl, :] = b1
        r2, g2, b2 = _lab_to_rgb_planes(clv, A, B)
        cl_ref[0, 0, sl, :] = r2
        cl_ref[0, 1, sl, :] = g2
        cl_ref[0, 2, sl, :] = b2
        return carry

    jax.lax.fori_loop(0, H // ROWS, apply_body, 0)


def kernel(x):
    N, C, H, W = x.shape
    eq, cl = pl.pallas_call(
        _wb_clahe_kernel,
        grid=(N,),
        in_specs=[pl.BlockSpec((1, 3, H, W), lambda n: (n, 0, 0, 0))],
        out_specs=[
            pl.BlockSpec((1, 3, H, W), lambda n: (n, 0, 0, 0)),
            pl.BlockSpec((1, 3, H, W), lambda n: (n, 0, 0, 0)),
        ],
        out_shape=[
            jax.ShapeDtypeStruct((N, C, H, W), jnp.float32),
            jax.ShapeDtypeStruct((N, C, H, W), jnp.float32),
        ],
        scratch_shapes=[
            pltpu.VMEM((H, W), jnp.float32),
            pltpu.VMEM((H, W), jnp.float32),
            pltpu.VMEM((H, W), jnp.float32),
            pltpu.VMEM((64, _NBINS), jnp.float32),
            pltpu.VMEM((72, _NBINS), jnp.float32),
        ],
        compiler_params=pltpu.CompilerParams(
            dimension_semantics=("parallel",),
            vmem_limit_bytes=56 * 1024 * 1024,
        ),
    )(x)
    return eq, cl
